# baseline (speedup 1.0000x reference)
import functools

import jax
import jax.numpy as jnp
from jax import lax
from jax.experimental import pallas as pl
from jax.experimental.pallas import tpu as pltpu
from jax.experimental.pallas import tpu_sc as plsc

L, T, D = 32, 1024, 1024
LANES = 16
NCHUNK = T // LANES
BIG = T


@functools.partial(
    pl.kernel,
    out_type=(),
    mesh=plsc.VectorSubcoreMesh(core_axis_name="c", subcore_axis_name="s",
                                num_cores=2, num_subcores=16),
    scratch_types=[
        pltpu.VMEM((LANES,), jnp.float32),
        pltpu.VMEM((LANES,), jnp.int32),
        pltpu.VMEM((LANES,), jnp.float32),
        pltpu.VMEM((T,), jnp.int32),
        pltpu.VMEM((T,), jnp.float32),
        pltpu.VMEM((D,), jnp.float32),
    ],
)
def _sc_update(layer_hbm, step_hbm, ec_hbm, ev_hbm, bev_ref, bstep_ref,
               bec_ref, lay_v, step_v, ec_v, row_v, ecrow_v, ev_v):
    cid = lax.axis_index("c")
    sid = lax.axis_index("s")
    is_owner = (cid == 0) & (sid == 0)

    @pl.when(is_owner)
    def _():
        pltpu.sync_copy(layer_hbm, lay_v)
        pltpu.sync_copy(step_hbm, step_v)
        pltpu.sync_copy(ec_hbm, ec_v)
        layer = lay_v[...][0].astype(jnp.int32)
        pltpu.sync_copy(bstep_ref.at[layer], row_v)
        pltpu.sync_copy(bec_ref.at[layer], ecrow_v)
        pltpu.sync_copy(ev_hbm, ev_v)

        iota = lax.iota(jnp.int32, LANES)
        iota_f = iota.astype(jnp.float32)
        bigf = jnp.float32(1e9)

        def body(i, carry):
            acc_occ, acc_emp = carry
            v = row_v[pl.ds(i * LANES, LANES)]
            gidx_f = iota_f + jnp.float32(i * LANES)
            enc = v.astype(jnp.float32) * jnp.float32(T) + gidx_f
            acc_occ = jnp.minimum(acc_occ, enc)
            acc_emp = jnp.minimum(acc_emp, jnp.where(v == -1, gidx_f, bigf))
            return acc_occ, acc_emp

        acc_occ, acc_emp = lax.fori_loop(
            0, NCHUNK, body,
            (jnp.full((LANES,), 1e9, jnp.float32),
             jnp.full((LANES,), 1e9, jnp.float32)))
        m_occ = acc_occ[0]
        m_emp = acc_emp[0]
        for j in range(1, LANES):
            m_occ = jnp.minimum(m_occ, acc_occ[j])
            m_emp = jnp.minimum(m_emp, acc_emp[j])
        slot_occ = m_occ.astype(jnp.int32) & (T - 1)
        slot = jnp.where(m_emp < bigf, m_emp.astype(jnp.int32), slot_occ)

        chunk = slot // LANES
        lane = slot - chunk * LANES
        off = chunk * LANES
        hit = iota == lane
        row_v[pl.ds(off, LANES)] = jnp.where(hit, step_v[...],
                                             row_v[pl.ds(off, LANES)])
        ecrow_v[pl.ds(off, LANES)] = jnp.where(hit, ec_v[...],
                                               ecrow_v[pl.ds(off, LANES)])

        pltpu.sync_copy(row_v, bstep_ref.at[layer])
        pltpu.sync_copy(ecrow_v, bec_ref.at[layer])
        pltpu.sync_copy(ev_v, bev_ref.at[layer, slot])


def kernel(layer, step, evidence, event_count, bank_evidence, bank_step,
           bank_event_count):
    lay16 = jnp.full((LANES,), layer, jnp.float32)
    step16 = jnp.full((LANES,), step, bank_step.dtype)
    ec16 = jnp.full((LANES,), event_count, bank_event_count.dtype)
    ev = evidence.astype(bank_evidence.dtype)

    bev_r = jax.new_ref(bank_evidence)
    bstep_r = jax.new_ref(bank_step)
    bec_r = jax.new_ref(bank_event_count)
    _sc_update(lay16, step16, ec16, ev, bev_r, bstep_r, bec_r)
    return bev_r[...], bstep_r[...], bec_r[...]
